# Initial kernel scaffold; baseline (speedup 1.0000x reference)
import jax
import jax.numpy as jnp
from jax.experimental import pallas as pl


def kernel(mem, val, write_idx, read_idx):
    raise NotImplementedError("write your pallas kernel here")


# speedup vs baseline: 1.8237x; 1.8237x over previous
import functools

import jax
import jax.numpy as jnp
from jax import lax
from jax.experimental import pallas as pl
from jax.experimental.pallas import tpu as pltpu
from jax.experimental.pallas import tpu_sc as plsc

_M = 100000
_D = 128
_B = 16384
_NW = 32
_SHARD = 3200
_TPAD = _NW * _SHARD
_CHUNK = _B // _NW
_NTRASH = 128

_mesh = plsc.VectorSubcoreMesh(core_axis_name="c", subcore_axis_name="s")
_params = pltpu.CompilerParams(needs_layout_passes=False)


def _worker_id():
    return lax.axis_index("s") * 2 + lax.axis_index("c")


@functools.partial(
    pl.kernel,
    out_type=jax.ShapeDtypeStruct((_TPAD,), jnp.int32),
    mesh=_mesh,
    compiler_params=_params,
    scratch_types=[
        pltpu.VMEM((_B,), jnp.int32),
        pltpu.VMEM((_SHARD,), jnp.int32),
    ],
)
def _build_table(widx_hbm, table_hbm, widx_v, tloc_v):
    wid = _worker_id()
    lo = wid * _SHARD

    pltpu.sync_copy(widx_hbm, widx_v)

    neg1 = jnp.full((16,), -1, jnp.int32)

    def init_body(i, carry):
        tloc_v[pl.ds(i * 16, 16)] = neg1
        return carry

    lax.fori_loop(0, _SHARD // 16, init_body, 0)

    iota = lax.iota(jnp.int32, 16)
    nxt = jnp.minimum(iota + 1, 15)
    is_last_lane = iota == 15

    def body(v, carry):
        w = widx_v[pl.ds(v * 16, 16)]
        j = jnp.full((16,), v * 16, jnp.int32) + iota
        k = w * _B + j
        ks, js = plsc.sort_key_val(k, j)
        ws = ks >> 14
        ws_next = ws.at[nxt].get(mode="promise_in_bounds")
        last = (ws != ws_next) | is_last_lane
        msk = last & (ws >= lo) & (ws < lo + _SHARD)
        plsc.store_scatter(tloc_v, [ws - lo], js, mask=msk)
        return carry

    lax.fori_loop(0, _B // 16, body, 0)

    pltpu.sync_copy(tloc_v, table_hbm.at[pl.ds(lo, _SHARD)])


@functools.partial(
    pl.kernel,
    out_type=jax.ShapeDtypeStruct((_B + _NTRASH, _D), jnp.float32),
    mesh=_mesh,
    compiler_params=_params,
    scratch_types=[
        pltpu.VMEM((_CHUNK // 128, 128), jnp.int32),
        pltpu.VMEM((_CHUNK // 128, 128), jnp.int32),
        pltpu.VMEM((_CHUNK // 128, 128), jnp.int32),
        pltpu.VMEM((_CHUNK // 128, 128), jnp.int32),
        pltpu.VMEM((_CHUNK, _D), jnp.float32),
        pltpu.SemaphoreType.DMA,
    ],
)
def _assemble(mem_hbm, val_hbm, ridx_hbm, table_hbm, out_hbm,
              r_v, s_v, g_v, d_v, buf_v, sem):
    wid = _worker_id()
    base = wid * _CHUNK
    nchunks = _CHUNK // 128

    for c in range(nchunks):
        pltpu.sync_copy(ridx_hbm.at[pl.ds(base + c * 128, 128)], r_v.at[c])

    for c in range(nchunks):
        pltpu.async_copy(table_hbm.at[r_v.at[c]], s_v.at[c], sem).wait()

    iota = lax.iota(jnp.int32, 16)
    for vi in range(_CHUNK // 16):
        c, o = divmod(vi, 8)
        s = s_v[c, pl.ds(o * 16, 16)]
        pos = jnp.full((16,), base + vi * 16, jnp.int32) + iota
        m = s >= 0
        g_v[c, pl.ds(o * 16, 16)] = jnp.where(m, s, pos)
        d_v[c, pl.ds(o * 16, 16)] = jnp.where(m, _B + (pos & (_NTRASH - 1)), pos)

    for c in range(nchunks):
        pltpu.async_copy(val_hbm.at[g_v.at[c]], buf_v.at[pl.ds(c * 128, 128)], sem).wait()
    pltpu.sync_copy(buf_v, out_hbm.at[pl.ds(base, _CHUNK)])

    for c in range(nchunks):
        pltpu.async_copy(mem_hbm.at[r_v.at[c]], buf_v.at[pl.ds(c * 128, 128)], sem).wait()
    for c in range(nchunks):
        pltpu.async_copy(buf_v.at[pl.ds(c * 128, 128)], out_hbm.at[d_v.at[c]], sem).wait()


def kernel(mem, val, write_idx, read_idx):
    table = _build_table(write_idx)
    out = _assemble(mem, val, read_idx, table)
    return out[:_B]


# speedup vs baseline: 2.5078x; 1.3751x over previous
import functools

import jax
import jax.numpy as jnp
from jax import lax
from jax.experimental import pallas as pl
from jax.experimental.pallas import tpu as pltpu
from jax.experimental.pallas import tpu_sc as plsc

_M = 100000
_D = 128
_B = 16384
_SHARD = 6400
_TPAD = 16 * _SHARD
_CHUNK = _B // 32
_NC = _CHUNK // 128

_mesh = plsc.VectorSubcoreMesh(core_axis_name="c", subcore_axis_name="s")
_params = pltpu.CompilerParams(needs_layout_passes=False)


@functools.partial(
    pl.kernel,
    out_type=jax.ShapeDtypeStruct((2 * _B, _D), jnp.float32),
    mesh=_mesh,
    compiler_params=_params,
    scratch_types=[
        pltpu.VMEM((_B,), jnp.int32),
        pltpu.VMEM((_SHARD,), jnp.int32),
        pltpu.VMEM_SHARED((_TPAD,), jnp.int32),
        pltpu.VMEM((_NC, 128), jnp.int32),
        pltpu.VMEM((_NC, 128), jnp.int32),
        pltpu.VMEM((_NC, 128), jnp.int32),
        pltpu.VMEM((_NC, 128), jnp.int32),
        pltpu.VMEM((_CHUNK, _D), jnp.float32),
        pltpu.SemaphoreType.DMA,
        pltpu.SemaphoreType.DMA,
    ],
)
def _fused(mem_hbm, val_hbm, widx_hbm, ridx_hbm, out_hbm,
           widx_v, tloc_v, tsh_v, r_v, s_v, g_v, d_v, buf_v, sem, sem2):
    sid = lax.axis_index("s")
    wid = sid * 2 + lax.axis_index("c")
    base = wid * _CHUNK
    lo = sid * _SHARD

    cp_w = pltpu.async_copy(widx_hbm, widx_v, sem)
    for c in range(_NC):
        pltpu.sync_copy(ridx_hbm.at[pl.ds(base + c * 128, 128)], r_v.at[c])
    mem_dmas = [
        pltpu.async_copy(mem_hbm.at[r_v.at[c]], buf_v.at[pl.ds(c * 128, 128)], sem2)
        for c in range(_NC)
    ]

    neg1 = jnp.full((16,), -1, jnp.int32)

    def init_body(i, carry):
        tloc_v[pl.ds(i * 16, 16)] = neg1
        return carry

    lax.fori_loop(0, _SHARD // 16, init_body, 0)

    cp_w.wait()
    iota = lax.iota(jnp.int32, 16)

    def body(v, carry):
        w = widx_v[pl.ds(v * 16, 16)]
        j = jnp.full((16,), v * 16, jnp.int32) + iota
        _, lastm = plsc.scan_count(w)
        msk = lastm & (w >= lo) & (w < lo + _SHARD)
        plsc.store_scatter(tloc_v, [w - lo], j, mask=msk)
        return carry

    lax.fori_loop(0, _B // 16, body, 0)

    pltpu.sync_copy(tloc_v, tsh_v.at[pl.ds(lo, _SHARD)])

    for h in mem_dmas:
        h.wait()
    pltpu.sync_copy(buf_v, out_hbm.at[pl.ds(base, _CHUNK)])

    plsc.subcore_barrier()

    for c in range(_NC):
        pltpu.async_copy(tsh_v.at[r_v.at[c]], s_v.at[c], sem).wait()

    for vi in range(_CHUNK // 16):
        c, o = divmod(vi, 8)
        s = s_v[c, pl.ds(o * 16, 16)]
        pos = jnp.full((16,), base + vi * 16, jnp.int32) + iota
        m = s >= 0
        g_v[c, pl.ds(o * 16, 16)] = jnp.where(m, s, pos)
        d_v[c, pl.ds(o * 16, 16)] = jnp.where(m, pos, _B + pos)

    val_dmas = [
        pltpu.async_copy(val_hbm.at[g_v.at[c]], buf_v.at[pl.ds(c * 128, 128)], sem2)
        for c in range(_NC)
    ]
    for h in val_dmas:
        h.wait()
    for c in range(_NC):
        pltpu.async_copy(buf_v.at[pl.ds(c * 128, 128)], out_hbm.at[d_v.at[c]], sem2).wait()


def kernel(mem, val, write_idx, read_idx):
    out = _fused(mem, val, write_idx, read_idx)
    return out[:_B]


# speedup vs baseline: 2.6474x; 1.0557x over previous
import functools

import jax
import jax.numpy as jnp
from jax import lax
from jax.experimental import pallas as pl
from jax.experimental.pallas import tpu as pltpu
from jax.experimental.pallas import tpu_sc as plsc

_M = 100000
_D = 128
_B = 16384
_SHARD = 6400
_TPAD = 16 * _SHARD
_CHUNK = _B // 32
_NC = _CHUNK // 128

_mesh = plsc.VectorSubcoreMesh(core_axis_name="c", subcore_axis_name="s")
_params = pltpu.CompilerParams(needs_layout_passes=False)


@functools.partial(
    pl.kernel,
    out_type=jax.ShapeDtypeStruct((_B, _D), jnp.float32),
    mesh=_mesh,
    compiler_params=_params,
    scratch_types=[
        pltpu.VMEM((_B,), jnp.int32),
        pltpu.VMEM((_SHARD,), jnp.int32),
        pltpu.VMEM_SHARED((_TPAD,), jnp.int32),
        pltpu.VMEM((_NC, 128), jnp.int32),
        pltpu.VMEM((_NC, 128), jnp.int32),
        pltpu.VMEM((_CHUNK,), jnp.int32),
        pltpu.VMEM((_CHUNK,), jnp.int32),
        pltpu.VMEM((_NC, 128), jnp.int32),
        pltpu.VMEM((_NC, 128), jnp.int32),
        pltpu.VMEM((_CHUNK, _D), jnp.float32),
        pltpu.VMEM((128, _D), jnp.float32),
        pltpu.SemaphoreType.DMA,
        pltpu.SemaphoreType.DMA,
    ],
)
def _fused(mem_hbm, val_hbm, widx_hbm, ridx2_hbm, out_hbm,
           widx_v, tloc_v, tsh_v, r_v, s_v, sc_v, pc_v, g2_v, d2_v,
           buf_v, vbuf_v, sem, sem2):
    sid = lax.axis_index("s")
    wid = sid * 2 + lax.axis_index("c")
    base = wid * _CHUNK
    lo = sid * _SHARD

    cp_w = pltpu.async_copy(widx_hbm, widx_v, sem)
    pltpu.sync_copy(ridx2_hbm.at[pl.ds(wid * _NC, _NC)], r_v)
    mem_dmas = [
        pltpu.async_copy(mem_hbm.at[r_v.at[c]], buf_v.at[pl.ds(c * 128, 128)], sem2)
        for c in range(_NC)
    ]

    neg1 = jnp.full((16,), -1, jnp.int32)

    def init_body(i, carry):
        tloc_v[pl.ds(i * 16, 16)] = neg1
        return carry

    lax.fori_loop(0, _SHARD // 16, init_body, 0)

    cp_w.wait()
    iota = lax.iota(jnp.int32, 16)

    def body(v, carry):
        w = widx_v[pl.ds(v * 16, 16)]
        j = jnp.full((16,), v * 16, jnp.int32) + iota
        _, lastm = plsc.scan_count(w)
        msk = lastm & (w >= lo) & (w < lo + _SHARD)
        plsc.store_scatter(tloc_v, [w - lo], j, mask=msk)
        return carry

    lax.fori_loop(0, _B // 16, body, 0)

    pltpu.sync_copy(tloc_v, tsh_v.at[pl.ds(lo, _SHARD)])

    for h in mem_dmas:
        h.wait()
    pltpu.sync_copy(buf_v, out_hbm.at[pl.ds(base, _CHUNK)])

    plsc.subcore_barrier()

    slot_dmas = [
        pltpu.async_copy(tsh_v.at[r_v.at[c]], s_v.at[c], sem)
        for c in range(_NC)
    ]
    for h in slot_dmas:
        h.wait()

    n = 0
    for vi in range(_CHUNK // 16):
        c, o = divmod(vi, 8)
        s = s_v[c, pl.ds(o * 16, 16)]
        pos = jnp.full((16,), base + vi * 16, jnp.int32) + iota
        m = s >= 0
        mi = jnp.where(m, 1, 0)
        idx = plsc.cumsum(mi) - 1 + n
        plsc.store_scatter(sc_v, [idx], s, mask=m)
        plsc.store_scatter(pc_v, [idx], pos, mask=m)
        n = n + jnp.sum(mi)

    zeros16 = jnp.full((16,), 0, jnp.int32)
    sv0 = sc_v[pl.ds(0, 16)]
    pv0 = pc_v[pl.ds(0, 16)]
    s_ent0 = sv0.at[zeros16].get(mode="promise_in_bounds")
    p_ent0 = pv0.at[zeros16].get(mode="promise_in_bounds")
    n_splat = jnp.full((16,), 0, jnp.int32) + n

    for q in range(_CHUNK // 16):
        c = q // 8
        o = q % 8
        lanes = jnp.full((16,), q * 16, jnp.int32) + iota
        valid = lanes < n_splat
        g2_v[c, pl.ds(o * 16, 16)] = jnp.where(valid, sc_v[pl.ds(q * 16, 16)], s_ent0)
        d2_v[c, pl.ds(o * 16, 16)] = jnp.where(valid, pc_v[pl.ds(q * 16, 16)], p_ent0)

    nchunk = (n + 127) >> 7

    for i in range(_NC):
        @pl.when(i < nchunk)
        def _():
            pltpu.async_copy(val_hbm.at[g2_v.at[i]], vbuf_v, sem2).wait()
            pltpu.async_copy(vbuf_v, out_hbm.at[d2_v.at[i]], sem2).wait()


def kernel(mem, val, write_idx, read_idx):
    return _fused(mem, val, write_idx, read_idx.reshape(_B // 128, 128))


# speedup vs baseline: 2.6555x; 1.0031x over previous
import functools

import jax
import jax.numpy as jnp
from jax import lax
from jax.experimental import pallas as pl
from jax.experimental.pallas import tpu as pltpu
from jax.experimental.pallas import tpu_sc as plsc

_M = 100000
_D = 128
_B = 16384
_SHARD = 6400
_TPAD = 16 * _SHARD
_CHUNK = _B // 32
_NC = _CHUNK // 128

_mesh = plsc.VectorSubcoreMesh(core_axis_name="c", subcore_axis_name="s")
_params = pltpu.CompilerParams(needs_layout_passes=False)


@functools.partial(
    pl.kernel,
    out_type=jax.ShapeDtypeStruct((_B, _D), jnp.float32),
    mesh=_mesh,
    compiler_params=_params,
    scratch_types=[
        pltpu.VMEM((_B,), jnp.int32),
        pltpu.VMEM((_SHARD,), jnp.int32),
        pltpu.VMEM_SHARED((_TPAD,), jnp.int32),
        pltpu.VMEM((_NC, 128), jnp.int32),
        pltpu.VMEM((_NC, 128), jnp.int32),
        pltpu.VMEM((_CHUNK,), jnp.int32),
        pltpu.VMEM((_CHUNK,), jnp.int32),
        pltpu.VMEM((_NC, 128), jnp.int32),
        pltpu.VMEM((_NC, 128), jnp.int32),
        pltpu.VMEM((_CHUNK, _D), jnp.float32),
        pltpu.VMEM((128, _D), jnp.float32),
        pltpu.SemaphoreType.DMA,
        pltpu.SemaphoreType.DMA,
    ],
)
def _fused(mem_hbm, val_hbm, widx_hbm, ridx2_hbm, out_hbm,
           widx_v, tloc_v, tsh_v, r_v, s_v, sc_v, pc_v, g2_v, d2_v,
           buf_v, vbuf_v, sem, sem2):
    sid = lax.axis_index("s")
    wid = sid * 2 + lax.axis_index("c")
    base = wid * _CHUNK
    lo = sid * _SHARD

    cp_w = pltpu.async_copy(widx_hbm, widx_v, sem)
    pltpu.sync_copy(ridx2_hbm.at[pl.ds(wid * _NC, _NC)], r_v)
    mem_dmas = [
        pltpu.async_copy(mem_hbm.at[r_v.at[c]], buf_v.at[pl.ds(c * 128, 128)], sem2)
        for c in range(_NC)
    ]

    neg1 = jnp.full((16,), -1, jnp.int32)

    def init_body(i, carry):
        tloc_v[pl.ds(i * 16, 16)] = neg1
        return carry

    lax.fori_loop(0, _SHARD // 16, init_body, 0)

    cp_w.wait()
    iota = lax.iota(jnp.int32, 16)
    shard_u = jnp.full((16,), _SHARD, jnp.uint32)

    def body(v8, carry):
        j0 = jnp.full((16,), v8 * 128, jnp.int32) + iota
        for k in range(8):
            w = widx_v[pl.ds(v8 * 128 + k * 16, 16)]
            u = w - lo
            _, lastm = plsc.scan_count(w)
            msk = lastm & (plsc.bitcast(u, jnp.uint32) < shard_u)
            plsc.store_scatter(tloc_v, [u], j0 + (k * 16), mask=msk)
        return carry

    lax.fori_loop(0, _B // 128, body, 0)

    pltpu.sync_copy(tloc_v, tsh_v.at[pl.ds(lo, _SHARD)])

    for h in mem_dmas:
        h.wait()
    pltpu.sync_copy(buf_v, out_hbm.at[pl.ds(base, _CHUNK)])

    plsc.subcore_barrier()

    slot_dmas = [
        pltpu.async_copy(tsh_v.at[r_v.at[c]], s_v.at[c], sem)
        for c in range(_NC)
    ]
    for h in slot_dmas:
        h.wait()

    n = 0
    for vi in range(_CHUNK // 16):
        c, o = divmod(vi, 8)
        s = s_v[c, pl.ds(o * 16, 16)]
        pos = jnp.full((16,), base + vi * 16, jnp.int32) + iota
        m = s >= 0
        mi = jnp.where(m, 1, 0)
        idx = plsc.cumsum(mi) - 1 + n
        plsc.store_scatter(sc_v, [idx], s, mask=m)
        plsc.store_scatter(pc_v, [idx], pos, mask=m)
        n = n + jnp.sum(mi)

    zeros16 = jnp.full((16,), 0, jnp.int32)
    sv0 = sc_v[pl.ds(0, 16)]
    pv0 = pc_v[pl.ds(0, 16)]
    s_ent0 = sv0.at[zeros16].get(mode="promise_in_bounds")
    p_ent0 = pv0.at[zeros16].get(mode="promise_in_bounds")
    n_splat = jnp.full((16,), 0, jnp.int32) + n

    for q in range(_CHUNK // 16):
        c = q // 8
        o = q % 8
        lanes = jnp.full((16,), q * 16, jnp.int32) + iota
        valid = lanes < n_splat
        g2_v[c, pl.ds(o * 16, 16)] = jnp.where(valid, sc_v[pl.ds(q * 16, 16)], s_ent0)
        d2_v[c, pl.ds(o * 16, 16)] = jnp.where(valid, pc_v[pl.ds(q * 16, 16)], p_ent0)

    nchunk = (n + 127) >> 7

    for i in range(_NC):
        @pl.when(i < nchunk)
        def _():
            pltpu.async_copy(val_hbm.at[g2_v.at[i]], vbuf_v, sem2).wait()
            pltpu.async_copy(vbuf_v, out_hbm.at[d2_v.at[i]], sem2).wait()


def kernel(mem, val, write_idx, read_idx):
    return _fused(mem, val, write_idx, read_idx.reshape(_B // 128, 128))


# speedup vs baseline: 3.5385x; 1.3325x over previous
import functools

import jax
import jax.numpy as jnp
from jax import lax
from jax.experimental import pallas as pl
from jax.experimental.pallas import tpu as pltpu
from jax.experimental.pallas import tpu_sc as plsc

_M = 100000
_D = 128
_B = 16384
_SHARD = 6400
_TPAD = 16 * _SHARD
_CHUNK = _B // 32
_NC = _CHUNK // 128

_mesh = plsc.VectorSubcoreMesh(core_axis_name="c", subcore_axis_name="s")
_params = pltpu.CompilerParams(needs_layout_passes=False)


@functools.partial(
    pl.kernel,
    out_type=jax.ShapeDtypeStruct((_B, _D), jnp.float32),
    mesh=_mesh,
    compiler_params=_params,
    scratch_types=[
        pltpu.VMEM((_B,), jnp.int32),
        pltpu.VMEM((_SHARD,), jnp.int32),
        pltpu.VMEM_SHARED((_TPAD,), jnp.int32),
        pltpu.VMEM((_NC, 128), jnp.int32),
        pltpu.VMEM((_NC, 128), jnp.int32),
        pltpu.VMEM((_CHUNK,), jnp.int32),
        pltpu.VMEM((_CHUNK,), jnp.int32),
        pltpu.VMEM((_NC, 128), jnp.int32),
        pltpu.VMEM((_NC, 128), jnp.int32),
        pltpu.VMEM((_CHUNK, _D), jnp.float32),
        pltpu.VMEM((128, _D), jnp.float32),
        pltpu.SemaphoreType.DMA,
        pltpu.SemaphoreType.DMA,
    ],
)
def _fused(mem_hbm, val_hbm, widx_hbm, ridx2_hbm, out_hbm,
           widx_v, tloc_v, tsh_v, r_v, s_v, sc_v, pc_v, g2_v, d2_v,
           buf_v, vbuf_v, sem, sem2):
    sid = lax.axis_index("s")
    wid = sid * 2 + lax.axis_index("c")
    base = wid * _CHUNK
    lo = sid * _SHARD

    cp_w = pltpu.async_copy(widx_hbm, widx_v, sem)
    pltpu.sync_copy(ridx2_hbm.at[pl.ds(wid * _NC, _NC)], r_v)
    mem_dmas = [
        pltpu.async_copy(mem_hbm.at[r_v.at[c]], buf_v.at[pl.ds(c * 128, 128)], sem2)
        for c in range(_NC)
    ]

    neg1 = jnp.full((16,), -1, jnp.int32)

    def init_body(i, carry):
        tloc_v[pl.ds(i * 16, 16)] = neg1
        return carry

    lax.fori_loop(0, _SHARD // 16, init_body, 0)

    cp_w.wait()
    iota = lax.iota(jnp.int32, 16)
    shard_u = jnp.full((16,), _SHARD, jnp.uint32)

    def body(v8, carry):
        j0 = jnp.full((16,), v8 * 128, jnp.int32) + iota
        ws = [widx_v[pl.ds(v8 * 128 + k * 16, 16)] for k in range(8)]
        lasts = [plsc.scan_count(w)[1] for w in ws]
        for k in range(8):
            u = ws[k] - lo
            msk = lasts[k] & (plsc.bitcast(u, jnp.uint32) < shard_u)
            plsc.store_scatter(tloc_v, [u], j0 + (k * 16), mask=msk)
        return carry

    lax.fori_loop(0, _B // 128, body, 0)

    pltpu.sync_copy(tloc_v, tsh_v.at[pl.ds(lo, _SHARD)])

    for h in mem_dmas:
        h.wait()
    pltpu.sync_copy(buf_v, out_hbm.at[pl.ds(base, _CHUNK)])

    plsc.subcore_barrier()

    slot_dmas = [
        pltpu.async_copy(tsh_v.at[r_v.at[c]], s_v.at[c], sem)
        for c in range(_NC)
    ]
    for h in slot_dmas:
        h.wait()

    n = 0
    for vi in range(_CHUNK // 16):
        c, o = divmod(vi, 8)
        s = s_v[c, pl.ds(o * 16, 16)]
        pos = jnp.full((16,), base + vi * 16, jnp.int32) + iota
        m = s >= 0
        mi = jnp.where(m, 1, 0)
        idx = plsc.cumsum(mi) - 1 + n
        plsc.store_scatter(sc_v, [idx], s, mask=m)
        plsc.store_scatter(pc_v, [idx], pos, mask=m)
        n = n + jnp.sum(mi)

    zeros16 = jnp.full((16,), 0, jnp.int32)
    sv0 = sc_v[pl.ds(0, 16)]
    pv0 = pc_v[pl.ds(0, 16)]
    s_ent0 = sv0.at[zeros16].get(mode="promise_in_bounds")
    p_ent0 = pv0.at[zeros16].get(mode="promise_in_bounds")
    n_splat = jnp.full((16,), 0, jnp.int32) + n

    for q in range(_CHUNK // 16):
        c = q // 8
        o = q % 8
        lanes = jnp.full((16,), q * 16, jnp.int32) + iota
        valid = lanes < n_splat
        g2_v[c, pl.ds(o * 16, 16)] = jnp.where(valid, sc_v[pl.ds(q * 16, 16)], s_ent0)
        d2_v[c, pl.ds(o * 16, 16)] = jnp.where(valid, pc_v[pl.ds(q * 16, 16)], p_ent0)

    nchunk = (n + 127) >> 7

    for i in range(_NC):
        @pl.when(i < nchunk)
        def _():
            pltpu.async_copy(val_hbm.at[g2_v.at[i]], vbuf_v, sem2).wait()
            pltpu.async_copy(vbuf_v, out_hbm.at[d2_v.at[i]], sem2).wait()


def kernel(mem, val, write_idx, read_idx):
    return _fused(mem, val, write_idx, read_idx.reshape(_B // 128, 128))


# speedup vs baseline: 3.6256x; 1.0246x over previous
import functools

import jax
import jax.numpy as jnp
from jax import lax
from jax.experimental import pallas as pl
from jax.experimental.pallas import tpu as pltpu
from jax.experimental.pallas import tpu_sc as plsc

_M = 100000
_D = 128
_B = 16384
_SHARD = 6400
_TPAD = 16 * _SHARD
_CHUNK = _B // 32
_NC = _CHUNK // 128

_mesh = plsc.VectorSubcoreMesh(core_axis_name="c", subcore_axis_name="s")
_params = pltpu.CompilerParams(needs_layout_passes=False)


@functools.partial(
    pl.kernel,
    out_type=jax.ShapeDtypeStruct((_B, _D), jnp.float32),
    mesh=_mesh,
    compiler_params=_params,
    scratch_types=[
        pltpu.VMEM((_B,), jnp.int32),
        pltpu.VMEM((_SHARD,), jnp.int32),
        pltpu.VMEM_SHARED((_TPAD,), jnp.int32),
        pltpu.VMEM((_NC, 128), jnp.int32),
        pltpu.VMEM((_NC, 128), jnp.int32),
        pltpu.VMEM((_CHUNK,), jnp.int32),
        pltpu.VMEM((_CHUNK,), jnp.int32),
        pltpu.VMEM((_NC, 128), jnp.int32),
        pltpu.VMEM((_NC, 128), jnp.int32),
        pltpu.VMEM((_CHUNK, _D), jnp.float32),
        pltpu.VMEM((128, _D), jnp.float32),
        pltpu.SemaphoreType.DMA,
        pltpu.SemaphoreType.DMA,
        pltpu.SemaphoreType.DMA,
    ],
)
def _fused(mem_hbm, val_hbm, widx_hbm, ridx2_hbm, out_hbm,
           widx_v, tloc_v, tsh_v, r_v, s_v, sc_v, pc_v, g2_v, d2_v,
           buf_v, vbuf_v, sem, sem2, sem3):
    sid = lax.axis_index("s")
    wid = sid * 2 + lax.axis_index("c")
    base = wid * _CHUNK
    lo = sid * _SHARD

    cp_w = pltpu.async_copy(widx_hbm, widx_v, sem)
    pltpu.sync_copy(ridx2_hbm.at[pl.ds(wid * _NC, _NC)], r_v)
    mem_dmas = [
        pltpu.async_copy(mem_hbm.at[r_v.at[c]], buf_v.at[pl.ds(c * 128, 128)], sem2)
        for c in range(_NC)
    ]

    neg1 = jnp.full((16,), -1, jnp.int32)

    def init_body(i, carry):
        tloc_v[pl.ds(i * 16, 16)] = neg1
        return carry

    lax.fori_loop(0, _SHARD // 16, init_body, 0)

    cp_w.wait()
    iota = lax.iota(jnp.int32, 16)
    shard_u = jnp.full((16,), _SHARD, jnp.uint32)

    def body(v8, carry):
        j0 = jnp.full((16,), v8 * 128, jnp.int32) + iota
        ws = [widx_v[pl.ds(v8 * 128 + k * 16, 16)] for k in range(8)]
        lasts = [plsc.scan_count(w)[1] for w in ws]
        for k in range(8):
            u = ws[k] - lo
            msk = lasts[k] & (plsc.bitcast(u, jnp.uint32) < shard_u)
            plsc.store_scatter(tloc_v, [u], j0 + (k * 16), mask=msk)
        return carry

    lax.fori_loop(0, _B // 128, body, 0)

    pltpu.sync_copy(tloc_v, tsh_v.at[pl.ds(lo, _SHARD)])

    for h in mem_dmas:
        h.wait()
    out_dma = pltpu.async_copy(buf_v, out_hbm.at[pl.ds(base, _CHUNK)], sem3)

    plsc.subcore_barrier()

    slot_dmas = [
        pltpu.async_copy(tsh_v.at[r_v.at[c]], s_v.at[c], sem)
        for c in range(_NC)
    ]
    for h in slot_dmas:
        h.wait()

    n = 0
    for g in range(_CHUNK // 128):
        ss = [s_v[g, pl.ds(o * 16, 16)] for o in range(8)]
        ms = [s >= 0 for s in ss]
        mis = [jnp.where(m, 1, 0) for m in ms]
        css = [plsc.cumsum(mi) for mi in mis]
        tots = [jnp.sum(mi) for mi in mis]
        for o in range(8):
            pos = jnp.full((16,), base + (g * 8 + o) * 16, jnp.int32) + iota
            idx = css[o] - 1 + n
            plsc.store_scatter(sc_v, [idx], ss[o], mask=ms[o])
            plsc.store_scatter(pc_v, [idx], pos, mask=ms[o])
            n = n + tots[o]

    zeros16 = jnp.full((16,), 0, jnp.int32)
    sv0 = sc_v[pl.ds(0, 16)]
    pv0 = pc_v[pl.ds(0, 16)]
    s_ent0 = sv0.at[zeros16].get(mode="promise_in_bounds")
    p_ent0 = pv0.at[zeros16].get(mode="promise_in_bounds")
    n_splat = jnp.full((16,), 0, jnp.int32) + n

    for q in range(_CHUNK // 16):
        c = q // 8
        o = q % 8
        lanes = jnp.full((16,), q * 16, jnp.int32) + iota
        valid = lanes < n_splat
        g2_v[c, pl.ds(o * 16, 16)] = jnp.where(valid, sc_v[pl.ds(q * 16, 16)], s_ent0)
        d2_v[c, pl.ds(o * 16, 16)] = jnp.where(valid, pc_v[pl.ds(q * 16, 16)], p_ent0)

    nchunk = (n + 127) >> 7
    out_dma.wait()

    for i in range(_NC):
        @pl.when(i < nchunk)
        def _():
            pltpu.async_copy(val_hbm.at[g2_v.at[i]], vbuf_v, sem2).wait()
            pltpu.async_copy(vbuf_v, out_hbm.at[d2_v.at[i]], sem2).wait()


def kernel(mem, val, write_idx, read_idx):
    return _fused(mem, val, write_idx, read_idx.reshape(_B // 128, 128))


# speedup vs baseline: 3.6335x; 1.0022x over previous
import functools

import jax
import jax.numpy as jnp
from jax import lax
from jax.experimental import pallas as pl
from jax.experimental.pallas import tpu as pltpu
from jax.experimental.pallas import tpu_sc as plsc

_M = 100000
_D = 128
_B = 16384
_SHARD = 6400
_TPAD = 16 * _SHARD
_CHUNK = _B // 32
_NC = _CHUNK // 128

_mesh = plsc.VectorSubcoreMesh(core_axis_name="c", subcore_axis_name="s")
_params = pltpu.CompilerParams(needs_layout_passes=False)


@functools.partial(
    pl.kernel,
    out_type=jax.ShapeDtypeStruct((_B, _D), jnp.float32),
    mesh=_mesh,
    compiler_params=_params,
    scratch_types=[
        pltpu.VMEM((_B,), jnp.int32),
        pltpu.VMEM((_SHARD,), jnp.int32),
        pltpu.VMEM_SHARED((_TPAD,), jnp.int32),
        pltpu.VMEM((_NC, 128), jnp.int32),
        pltpu.VMEM((_NC, 128), jnp.int32),
        pltpu.VMEM((_CHUNK,), jnp.int32),
        pltpu.VMEM((_CHUNK,), jnp.int32),
        pltpu.VMEM((_NC, 128), jnp.int32),
        pltpu.VMEM((_NC, 128), jnp.int32),
        pltpu.VMEM((_CHUNK, _D), jnp.float32),
        pltpu.VMEM((128, _D), jnp.float32),
        pltpu.SemaphoreType.DMA,
        pltpu.SemaphoreType.DMA,
        pltpu.SemaphoreType.DMA,
        pltpu.SemaphoreType.DMA((_NC,)),
    ],
)
def _fused(mem_hbm, val_hbm, widx_hbm, ridx2_hbm, out_hbm,
           widx_v, tloc_v, tsh_v, r_v, s_v, sc_v, pc_v, g2_v, d2_v,
           buf_v, vbuf_v, sem, sem2, sem3, sem4):
    sid = lax.axis_index("s")
    wid = sid * 2 + lax.axis_index("c")
    base = wid * _CHUNK
    lo = sid * _SHARD

    cp_w = pltpu.async_copy(widx_hbm, widx_v, sem)
    pltpu.sync_copy(ridx2_hbm.at[pl.ds(wid * _NC, _NC)], r_v)
    mem_dmas = [
        pltpu.async_copy(mem_hbm.at[r_v.at[c]], buf_v.at[pl.ds(c * 128, 128)], sem2)
        for c in range(_NC)
    ]

    neg1 = jnp.full((16,), -1, jnp.int32)

    def init_body(i, carry):
        tloc_v[pl.ds(i * 16, 16)] = neg1
        return carry

    lax.fori_loop(0, _SHARD // 16, init_body, 0)

    cp_w.wait()
    iota = lax.iota(jnp.int32, 16)
    shard_u = jnp.full((16,), _SHARD, jnp.uint32)

    def body(v8, carry):
        j0 = jnp.full((16,), v8 * 128, jnp.int32) + iota
        ws = [widx_v[pl.ds(v8 * 128 + k * 16, 16)] for k in range(8)]
        lasts = [plsc.scan_count(w)[1] for w in ws]
        for k in range(8):
            u = ws[k] - lo
            msk = lasts[k] & (plsc.bitcast(u, jnp.uint32) < shard_u)
            plsc.store_scatter(tloc_v, [u], j0 + (k * 16), mask=msk)
        return carry

    lax.fori_loop(0, _B // 128, body, 0)

    pub_dma = pltpu.async_copy(tloc_v, tsh_v.at[pl.ds(lo, _SHARD)], sem)

    for h in mem_dmas:
        h.wait()
    out_dma = pltpu.async_copy(buf_v, out_hbm.at[pl.ds(base, _CHUNK)], sem3)

    pub_dma.wait()
    plsc.subcore_barrier()

    slot_dmas = [
        pltpu.async_copy(tsh_v.at[r_v.at[c]], s_v.at[c], sem4.at[c])
        for c in range(_NC)
    ]

    n = 0
    for g in range(_CHUNK // 128):
        slot_dmas[g].wait()
        ss = [s_v[g, pl.ds(o * 16, 16)] for o in range(8)]
        ms = [s >= 0 for s in ss]
        mis = [jnp.where(m, 1, 0) for m in ms]
        css = [plsc.cumsum(mi) for mi in mis]
        tots = [jnp.sum(mi) for mi in mis]
        for o in range(8):
            pos = jnp.full((16,), base + (g * 8 + o) * 16, jnp.int32) + iota
            idx = css[o] - 1 + n
            plsc.store_scatter(sc_v, [idx], ss[o], mask=ms[o])
            plsc.store_scatter(pc_v, [idx], pos, mask=ms[o])
            n = n + tots[o]

    zeros16 = jnp.full((16,), 0, jnp.int32)
    sv0 = sc_v[pl.ds(0, 16)]
    pv0 = pc_v[pl.ds(0, 16)]
    s_ent0 = sv0.at[zeros16].get(mode="promise_in_bounds")
    p_ent0 = pv0.at[zeros16].get(mode="promise_in_bounds")
    n_splat = jnp.full((16,), 0, jnp.int32) + n

    for q in range(_CHUNK // 16):
        c = q // 8
        o = q % 8
        lanes = jnp.full((16,), q * 16, jnp.int32) + iota
        valid = lanes < n_splat
        g2_v[c, pl.ds(o * 16, 16)] = jnp.where(valid, sc_v[pl.ds(q * 16, 16)], s_ent0)
        d2_v[c, pl.ds(o * 16, 16)] = jnp.where(valid, pc_v[pl.ds(q * 16, 16)], p_ent0)

    nchunk = (n + 127) >> 7
    out_dma.wait()

    for i in range(_NC):
        @pl.when(i < nchunk)
        def _():
            pltpu.async_copy(val_hbm.at[g2_v.at[i]], vbuf_v, sem2).wait()
            pltpu.async_copy(vbuf_v, out_hbm.at[d2_v.at[i]], sem2).wait()


def kernel(mem, val, write_idx, read_idx):
    return _fused(mem, val, write_idx, read_idx.reshape(_B // 128, 128))
